# baseline (speedup 1.0000x reference)
import functools

import jax
import jax.numpy as jnp
from jax import lax
from jax.experimental import pallas as pl
from jax.experimental.pallas import tpu as pltpu
from jax.experimental.pallas import tpu_sc as plsc


def _make_sc_broadcast(nb, seq_len, dim, dtype):
    info = plsc.get_sparse_core_info()
    nw = info.num_cores * info.num_subcores
    rows_per_w = seq_len // nw
    chunk = 32
    n_chunks = rows_per_w // chunk

    mesh = plsc.VectorSubcoreMesh(core_axis_name="c", subcore_axis_name="s")

    @functools.partial(
        pl.kernel,
        mesh=mesh,
        out_type=jax.ShapeDtypeStruct((nb, seq_len, dim), dtype),
        scratch_types=[
            pltpu.VMEM((chunk, dim), dtype),
            pltpu.VMEM((chunk, dim), dtype),
            pltpu.SemaphoreType.DMA,
            pltpu.SemaphoreType.DMA,
            pltpu.SemaphoreType.DMA,
            pltpu.SemaphoreType.DMA,
        ],
    )
    def k(table_hbm, out_hbm, buf0, buf1, sg0, sg1, sw0, sw1):
        bufs, sgs, sws = [buf0, buf1], [sg0, sg1], [sw0, sw1]
        wid = lax.axis_index("s") * info.num_cores + lax.axis_index("c")
        base = wid * rows_per_w

        gathers = [None] * n_chunks
        writes = [None] * n_chunks

        def issue_writes(i):
            r = base + i * chunk
            writes[i] = [
                pltpu.async_copy(bufs[i % 2], out_hbm.at[b, pl.ds(r, chunk)],
                                 sws[i % 2])
                for b in range(nb)
            ]

        for i in range(n_chunks):
            if i >= 2:
                for c in writes[i - 2]:
                    c.wait()
            r = base + i * chunk
            gathers[i] = pltpu.async_copy(
                table_hbm.at[pl.ds(r, chunk)], bufs[i % 2], sgs[i % 2])
            if i >= 1:
                gathers[i - 1].wait()
                issue_writes(i - 1)

        gathers[n_chunks - 1].wait()
        issue_writes(n_chunks - 1)
        for i in (n_chunks - 2, n_chunks - 1):
            for c in writes[i]:
                c.wait()

    return k


def _tc_broadcast(nb, seq_len, dim, dtype, table):
    S = 1024

    def body(tab_ref, out_ref):
        out_ref[...] = jnp.broadcast_to(tab_ref[...][None], (nb, S, dim))

    return pl.pallas_call(
        body,
        grid=(seq_len // S,),
        in_specs=[pl.BlockSpec((S, dim), lambda i: (i, 0))],
        out_specs=pl.BlockSpec((nb, S, dim), lambda i: (0, i, 0)),
        out_shape=jax.ShapeDtypeStruct((nb, seq_len, dim), dtype),
    )(table)


def kernel(x, symbol_library):
    batch, seq_len, dim = x.shape
    dtype = symbol_library.dtype
    nb_sc = 1
    sc_out = _make_sc_broadcast(nb_sc, seq_len, dim, dtype)(symbol_library)
    tc_out = _tc_broadcast(batch - nb_sc, seq_len, dim, dtype, symbol_library)
    return jnp.concatenate([sc_out, tc_out], axis=0)


# speedup vs baseline: 3.0707x; 3.0707x over previous
import jax
import jax.numpy as jnp
from jax.experimental import pallas as pl
from jax.experimental.pallas import tpu as pltpu


def kernel(x, symbol_library):
    batch, seq_len, dim = x.shape
    dtype = symbol_library.dtype
    S = 1024
    n_chunks = seq_len // S

    def body(tab_hbm, out_hbm, buf0, buf1, sg0, sg1, sw0, sw1):
        bufs, sgs, sws = [buf0, buf1], [sg0, sg1], [sw0, sw1]
        gathers = [None] * n_chunks
        writes = [None] * n_chunks

        def issue_writes(i):
            writes[i] = [
                pltpu.make_async_copy(
                    bufs[i % 2], out_hbm.at[b, pl.ds(i * S, S), :], sws[i % 2])
                for b in range(batch)
            ]
            for c in writes[i]:
                c.start()

        for i in range(n_chunks):
            if i >= 2:
                for c in writes[i - 2]:
                    c.wait()
            gathers[i] = pltpu.make_async_copy(
                tab_hbm.at[pl.ds(i * S, S), :], bufs[i % 2], sgs[i % 2])
            gathers[i].start()
            if i >= 1:
                gathers[i - 1].wait()
                issue_writes(i - 1)

        gathers[n_chunks - 1].wait()
        issue_writes(n_chunks - 1)
        for i in (n_chunks - 2, n_chunks - 1):
            for c in writes[i]:
                c.wait()

    return pl.pallas_call(
        body,
        in_specs=[pl.BlockSpec(memory_space=pl.ANY)],
        out_specs=pl.BlockSpec(memory_space=pl.ANY),
        out_shape=jax.ShapeDtypeStruct((batch, seq_len, dim), dtype),
        scratch_shapes=[
            pltpu.VMEM((S, dim), dtype),
            pltpu.VMEM((S, dim), dtype),
            pltpu.SemaphoreType.DMA,
            pltpu.SemaphoreType.DMA,
            pltpu.SemaphoreType.DMA,
            pltpu.SemaphoreType.DMA,
        ],
    )(symbol_library)
